# baseline (speedup 1.0000x reference)
import functools

import jax
import jax.numpy as jnp
from jax import lax
from jax.experimental import pallas as pl
from jax.experimental.pallas import tpu as pltpu
from jax.experimental.pallas import tpu_sc as plsc

BATCH = 16384
EMBED_DIM = 128
NC = 2
NS = 16
NW = NC * NS
B_PER_W = BATCH // NW
CHUNK = 128
NCHUNKS = B_PER_W // CHUNK
SEG = EMBED_DIM // 16


def _mf_kernel(uid_hbm, aid_hbm, ue_hbm, ae_hbm, ub_hbm, ab_hbm, gb_hbm,
               out_hbm,
               uidx_v, aidx_v, uhi_v, ahi_v, ue_buf, ae_buf, ub_buf, ab_buf,
               out_buf, gb_v,
               sem0, sem1):
    wid = lax.axis_index("s") * NC + lax.axis_index("c")
    base = wid * B_PER_W
    sems = (sem0, sem1)

    pltpu.sync_copy(uid_hbm.at[pl.ds(wid * NCHUNKS, NCHUNKS)], uidx_v)
    pltpu.sync_copy(aid_hbm.at[pl.ds(wid * NCHUNKS, NCHUNKS)], aidx_v)
    pltpu.sync_copy(gb_hbm, gb_v)

    for c in range(NCHUNKS):
        for j in range(CHUNK // 16):
            sl = pl.ds(j * 16, 16)
            uhi_v[c, sl] = lax.shift_right_logical(uidx_v[c, sl], 4)
            ahi_v[c, sl] = lax.shift_right_logical(aidx_v[c, sl], 4)

    def start(c):
        slot = c % 2
        sem = sems[slot]
        ui = uidx_v.at[c]
        ai = aidx_v.at[c]
        return [
            pltpu.async_copy(ue_hbm.at[ui], ue_buf.at[slot], sem),
            pltpu.async_copy(ae_hbm.at[ai], ae_buf.at[slot], sem),
            pltpu.async_copy(ub_hbm.at[uhi_v.at[c]], ub_buf.at[slot], sem),
            pltpu.async_copy(ab_hbm.at[ahi_v.at[c]], ab_buf.at[slot], sem),
        ]

    def compute_chunk(c):
        slot = c % 2
        iota = lax.iota(jnp.int32, 16)
        gb = gb_v[...]

        def group_body(g, carry):
            row0 = g * 16
            tot = gb
            for r in range(16):
                row = row0 + r
                acc = (ue_buf[slot, row, pl.ds(0, 16)] *
                       ae_buf[slot, row, pl.ds(0, 16)])
                for s in range(1, SEG):
                    acc = acc + (ue_buf[slot, row, pl.ds(s * 16, 16)] *
                                 ae_buf[slot, row, pl.ds(s * 16, 16)])
                dot = jnp.sum(acc)
                tot = jnp.where(iota == r, dot, tot)
            ridx = iota + row0
            urem = jnp.bitwise_and(uidx_v[c, pl.ds(row0, 16)], 15)
            arem = jnp.bitwise_and(aidx_v[c, pl.ds(row0, 16)], 15)
            tot = tot + plsc.load_gather(ub_buf.at[slot], [ridx, urem])
            tot = tot + plsc.load_gather(ab_buf.at[slot], [ridx, arem])
            out_buf[pl.ds(c * CHUNK + row0, 16)] = tot
            return carry

        lax.fori_loop(0, CHUNK // 16, group_body, 0)

    copies = start(0)
    for c in range(NCHUNKS):
        nxt = start(c + 1) if c + 1 < NCHUNKS else None
        for cp in copies:
            cp.wait()
        compute_chunk(c)
        copies = nxt

    pltpu.sync_copy(out_buf, out_hbm.at[pl.ds(base, B_PER_W)])


def kernel(user_id, anime_id, user_embedding, anime_embedding, user_bias,
           anime_bias, global_bias):
    mesh = plsc.VectorSubcoreMesh(core_axis_name="c", subcore_axis_name="s")
    run = functools.partial(
        pl.kernel,
        mesh=mesh,
        compiler_params=pltpu.CompilerParams(
            needs_layout_passes=False, use_tc_tiling_on_sc=False),
        out_type=jax.ShapeDtypeStruct((BATCH,), jnp.float32),
        scratch_types=[
            pltpu.VMEM((NCHUNKS, CHUNK), jnp.int32),
            pltpu.VMEM((NCHUNKS, CHUNK), jnp.int32),
            pltpu.VMEM((NCHUNKS, CHUNK), jnp.int32),
            pltpu.VMEM((NCHUNKS, CHUNK), jnp.int32),
            pltpu.VMEM((2, CHUNK, EMBED_DIM), jnp.float32),
            pltpu.VMEM((2, CHUNK, EMBED_DIM), jnp.float32),
            pltpu.VMEM((2, CHUNK, 16), jnp.float32),
            pltpu.VMEM((2, CHUNK, 16), jnp.float32),
            pltpu.VMEM((B_PER_W,), jnp.float32),
            pltpu.VMEM((16,), jnp.float32),
            pltpu.SemaphoreType.DMA,
            pltpu.SemaphoreType.DMA,
        ],
    )(_mf_kernel)
    uid2d = user_id.astype(jnp.int32).reshape(BATCH // CHUNK, CHUNK)
    aid2d = anime_id.astype(jnp.int32).reshape(BATCH // CHUNK, CHUNK)
    ub_blk = user_bias.T[0].reshape(-1, 16)
    ab_blk = anime_bias.T[0].reshape(-1, 16)
    return run(uid2d, aid2d,
               user_embedding, anime_embedding, ub_blk, ab_blk,
               jnp.broadcast_to(global_bias, (16,)))


# speedup vs baseline: 1.0002x; 1.0002x over previous
import functools

import jax
import jax.numpy as jnp
from jax import lax
from jax.experimental import pallas as pl
from jax.experimental.pallas import tpu as pltpu
from jax.experimental.pallas import tpu_sc as plsc

BATCH = 16384
EMBED_DIM = 128
NC = 2
NS = 16
NW = NC * NS
B_PER_W = BATCH // NW
CHUNK = 128
NCHUNKS = B_PER_W // CHUNK
SEG = EMBED_DIM // 16


def _mf_kernel(uid_hbm, aid_hbm, ue_hbm, ae_hbm, ub_hbm, ab_hbm, gb_hbm,
               out_hbm,
               uidx_v, aidx_v, uhi_v, ahi_v, ue_buf, ae_buf, ub_buf, ab_buf,
               out_buf, gb_v,
               sem0, sem1):
    wid = lax.axis_index("s") * NC + lax.axis_index("c")
    base = wid * B_PER_W
    sems = (sem0, sem1)

    pltpu.sync_copy(uid_hbm.at[pl.ds(wid * NCHUNKS, NCHUNKS)], uidx_v)
    pltpu.sync_copy(aid_hbm.at[pl.ds(wid * NCHUNKS, NCHUNKS)], aidx_v)
    pltpu.sync_copy(gb_hbm, gb_v)

    for c in range(NCHUNKS):
        for j in range(CHUNK // 16):
            sl = pl.ds(j * 16, 16)
            uhi_v[c, sl] = lax.shift_right_logical(uidx_v[c, sl], 4)
            ahi_v[c, sl] = lax.shift_right_logical(aidx_v[c, sl], 4)

    def start(c):
        slot = c % 2
        sem = sems[slot]
        ui = uidx_v.at[c]
        ai = aidx_v.at[c]
        return [
            pltpu.async_copy(ue_hbm.at[ui], ue_buf.at[slot], sem),
            pltpu.async_copy(ae_hbm.at[ai], ae_buf.at[slot], sem),
            pltpu.async_copy(ub_hbm.at[uhi_v.at[c]], ub_buf.at[slot], sem),
            pltpu.async_copy(ab_hbm.at[ahi_v.at[c]], ab_buf.at[slot], sem),
        ]

    def compute_chunk(c):
        slot = c % 2
        iota = lax.iota(jnp.int32, 16)
        gb = gb_v[...]

        def group_body(g, carry):
            row0 = g * 16
            tot = gb
            for r in range(16):
                row = row0 + r
                acc = (ue_buf[slot, row, pl.ds(0, 16)] *
                       ae_buf[slot, row, pl.ds(0, 16)])
                for s in range(1, SEG):
                    acc = acc + (ue_buf[slot, row, pl.ds(s * 16, 16)] *
                                 ae_buf[slot, row, pl.ds(s * 16, 16)])
                dot = jnp.sum(acc)
                tot = jnp.where(iota == r, dot, tot)
            ridx = iota + row0
            urem = jnp.bitwise_and(uidx_v[c, pl.ds(row0, 16)], 15)
            arem = jnp.bitwise_and(aidx_v[c, pl.ds(row0, 16)], 15)
            tot = tot + plsc.load_gather(ub_buf.at[slot], [ridx, urem])
            tot = tot + plsc.load_gather(ab_buf.at[slot], [ridx, arem])
            out_buf[pl.ds(c * CHUNK + row0, 16)] = tot
            return carry

        lax.fori_loop(0, CHUNK // 16, group_body, 0)

    copies = start(0)
    for c in range(NCHUNKS):
        nxt = start(c + 1) if c + 1 < NCHUNKS else None
        for cp in copies:
            cp.wait()
        compute_chunk(c)
        copies = nxt

    pltpu.sync_copy(out_buf, out_hbm.at[pl.ds(base, B_PER_W)])


def kernel(user_id, anime_id, user_embedding, anime_embedding, user_bias,
           anime_bias, global_bias):
    mesh = plsc.VectorSubcoreMesh(core_axis_name="c", subcore_axis_name="s")
    run = functools.partial(
        pl.kernel,
        mesh=mesh,
        compiler_params=pltpu.CompilerParams(
            needs_layout_passes=False, use_tc_tiling_on_sc=False),
        out_type=jax.ShapeDtypeStruct((BATCH,), jnp.float32),
        scratch_types=[
            pltpu.VMEM((NCHUNKS, CHUNK), jnp.int32),
            pltpu.VMEM((NCHUNKS, CHUNK), jnp.int32),
            pltpu.VMEM((NCHUNKS, CHUNK), jnp.int32),
            pltpu.VMEM((NCHUNKS, CHUNK), jnp.int32),
            pltpu.VMEM((2, CHUNK, EMBED_DIM), jnp.float32),
            pltpu.VMEM((2, CHUNK, EMBED_DIM), jnp.float32),
            pltpu.VMEM((2, CHUNK, 16), jnp.float32),
            pltpu.VMEM((2, CHUNK, 16), jnp.float32),
            pltpu.VMEM((B_PER_W,), jnp.float32),
            pltpu.VMEM((16,), jnp.float32),
            pltpu.SemaphoreType.DMA,
            pltpu.SemaphoreType.DMA,
        ],
    )(_mf_kernel)
    uid2d = user_id.astype(jnp.int32).reshape(BATCH // CHUNK, CHUNK)
    aid2d = anime_id.astype(jnp.int32).reshape(BATCH // CHUNK, CHUNK)
    ub_blk = jax.lax.optimization_barrier(user_bias.T).reshape(-1, 16)
    ab_blk = jax.lax.optimization_barrier(anime_bias.T).reshape(-1, 16)
    return run(uid2d, aid2d,
               user_embedding, anime_embedding, ub_blk, ab_blk,
               jnp.broadcast_to(global_bias, (16,)))
